# baseline (speedup 1.0000x reference)
import functools

import jax
import jax.numpy as jnp
from jax import lax
from jax.experimental import pallas as pl
from jax.experimental.pallas import tpu as pltpu
from jax.experimental.pallas import tpu_sc as plsc

N = 10000
E = 320000
D_IN = 128
D_OUT = 64
N_HOP = 3

NC = 2
NS = 16
NW = NC * NS
EPW = E // NW
CHUNK = 128
NFULL = EPW // CHUNK
TAIL = EPW - NFULL * CHUNK
NPAD = 10240
STRIPE = NPAD // NS
WPAD = 64
RB = 512
NRB = NPAD // RB
NBUF = 6
LOOKA = 3

_mesh = plsc.VectorSubcoreMesh(core_axis_name="c", subcore_axis_name="s")


@functools.partial(
    pl.kernel,
    out_type=jax.ShapeDtypeStruct((NC * N_HOP * NPAD,), jnp.float32),
    mesh=_mesh,
    compiler_params=pltpu.CompilerParams(use_tc_tiling_on_sc=False),
    scratch_types=[
        pltpu.VMEM((EPW,), jnp.int32),
        pltpu.VMEM((CHUNK,), jnp.float32),
        pltpu.VMEM((STRIPE,), jnp.float32),
        pltpu.VMEM_SHARED((NPAD,), jnp.float32),
        pltpu.VMEM_SHARED((NPAD,), jnp.float32),
        pltpu.VMEM_SHARED((NPAD,), jnp.float32),
    ],
)
def _deg_kernel(edge0, edge1, edge2, degp_hbm, idx_v, ones_v, zeros_v,
                acc0, acc1, acc2):
    c = lax.axis_index("c")
    s = lax.axis_index("s")
    wid = s * NC + c
    edges = (edge0, edge1, edge2)
    accs = (acc0, acc1, acc2)
    for i in range(CHUNK // 16):
        ones_v[pl.ds(i * 16, 16)] = jnp.ones((16,), jnp.float32)

    def _zero_body(i, carry):
        zeros_v[pl.ds(i * 16, 16)] = jnp.zeros((16,), jnp.float32)
        return carry

    lax.fori_loop(0, STRIPE // 16, _zero_body, 0)
    for j in range(N_HOP):
        pltpu.sync_copy(zeros_v, accs[j].at[pl.ds(s * STRIPE, STRIPE)])
    plsc.subcore_barrier()

    for j in range(N_HOP):
        pltpu.sync_copy(edges[j].at[1, pl.ds(wid * EPW, EPW)], idx_v)

        def _body(ch, carry):
            pltpu.sync_copy(ones_v,
                            accs[j].at[idx_v.at[pl.ds(ch * CHUNK, CHUNK)]],
                            add=True)
            return carry

        lax.fori_loop(0, NFULL, _body, 0)
        pltpu.sync_copy(ones_v.at[pl.ds(0, TAIL)],
                        accs[j].at[idx_v.at[pl.ds(NFULL * CHUNK, TAIL)]],
                        add=True)
    plsc.subcore_barrier()
    for j in range(N_HOP):
        pltpu.sync_copy(
            accs[j].at[pl.ds(s * STRIPE, STRIPE)],
            degp_hbm.at[pl.ds((c * N_HOP + j) * NPAD + s * STRIPE, STRIPE)])


@functools.partial(
    pl.kernel,
    out_type=jax.ShapeDtypeStruct((NC * NPAD, WPAD), jnp.float32),
    mesh=_mesh,
    compiler_params=pltpu.CompilerParams(use_tc_tiling_on_sc=False),
    scratch_types=[
        pltpu.VMEM((EPW,), jnp.int32),
        pltpu.VMEM((EPW,), jnp.int32),
        [pltpu.VMEM((CHUNK, WPAD), jnp.float32) for _ in range(NBUF)],
        pltpu.VMEM((128, WPAD), jnp.float32),
        pltpu.VMEM_SHARED((NPAD, WPAD), jnp.float32),
        [pltpu.SemaphoreType.DMA for _ in range(NBUF)],
        [pltpu.SemaphoreType.DMA for _ in range(NBUF)],
    ],
)
def _agg_one(y_hbm, edge, aggp_hbm, sidx, didx, gbufs, zrow, acc,
             gsems, ssems):
    c = lax.axis_index("c")
    s = lax.axis_index("s")
    wid = s * NC + c

    def _zero_body(i, carry):
        for k in range(WPAD // 16):
            zrow[i, pl.ds(k * 16, 16)] = jnp.zeros((16,), jnp.float32)
        return carry

    lax.fori_loop(0, 128, _zero_body, 0)

    for t in range(STRIPE // 128):
        pltpu.sync_copy(zrow, acc.at[pl.ds(s * STRIPE + t * 128, 128)])
    pltpu.sync_copy(edge.at[0, pl.ds(wid * EPW, EPW)], sidx)
    pltpu.sync_copy(edge.at[1, pl.ds(wid * EPW, EPW)], didx)
    plsc.subcore_barrier()

    for b in range(LOOKA):
        pltpu.async_copy(y_hbm.at[sidx.at[pl.ds(b * CHUNK, CHUNK)]],
                         gbufs[b], gsems[b])

    def _group(g, carry):
        for b in range(NBUF):
            ch = g * NBUF + b
            ch_pf = ch + LOOKA
            slot_pf = (b + LOOKA) % NBUF

            @pl.when(jnp.logical_and(ch_pf >= NBUF, ch_pf < NFULL))
            def _drain():
                pltpu.make_async_copy(gbufs[slot_pf],
                                      acc.at[pl.ds(0, CHUNK)],
                                      ssems[slot_pf]).wait()

            @pl.when(ch_pf < NFULL)
            def _prefetch():
                pltpu.async_copy(
                    y_hbm.at[sidx.at[pl.ds(ch_pf * CHUNK, CHUNK)]],
                    gbufs[slot_pf], gsems[slot_pf])

            pltpu.make_async_copy(y_hbm.at[pl.ds(0, CHUNK)],
                                  gbufs[b], gsems[b]).wait()
            pltpu.async_copy(gbufs[b],
                             acc.at[didx.at[pl.ds(ch * CHUNK, CHUNK)]],
                             ssems[b], add=True)
        return carry

    lax.fori_loop(0, NFULL // NBUF, _group, 0)
    for b in range(NBUF):
        pltpu.make_async_copy(gbufs[b], acc.at[pl.ds(0, CHUNK)],
                              ssems[b]).wait()
    pltpu.async_copy(
        y_hbm.at[sidx.at[pl.ds(NFULL * CHUNK, TAIL)]],
        gbufs[0].at[pl.ds(0, TAIL)], gsems[0]).wait()
    pltpu.sync_copy(gbufs[0].at[pl.ds(0, TAIL)],
                    acc.at[didx.at[pl.ds(NFULL * CHUNK, TAIL)]],
                    add=True)
    plsc.subcore_barrier()
    pltpu.sync_copy(
        acc.at[pl.ds(s * STRIPE, STRIPE)],
        aggp_hbm.at[pl.ds(c * NPAD + s * STRIPE, STRIPE)])


def _mm1_body(bb1_r, bb2_r, bb3_r, wstack_r, biascat_r, xw_r):
    xw_r[...] = (
        jnp.dot(bb1_r[...], wstack_r[0], preferred_element_type=jnp.float32)
        + jnp.dot(bb2_r[...], wstack_r[1], preferred_element_type=jnp.float32)
        + jnp.dot(bb3_r[...], wstack_r[2], preferred_element_type=jnp.float32)
        + biascat_r[...])


def _mm1_call(bb1, bb2, bb3, wstack, biascat):
    blk = pl.BlockSpec((RB, D_IN), lambda i: (i, 0))
    return pl.pallas_call(
        _mm1_body,
        grid=(NRB,),
        in_specs=[blk, blk, blk,
                  pl.BlockSpec(wstack.shape, lambda i: (0, 0, 0)),
                  pl.BlockSpec(biascat.shape, lambda i: (0, 0))],
        out_specs=pl.BlockSpec((RB, (N_HOP + 1) * D_OUT), lambda i: (i, 0)),
        out_shape=jax.ShapeDtypeStruct((N, (N_HOP + 1) * D_OUT), jnp.float32),
    )(bb1, bb2, bb3, wstack, biascat)


def _mm2_body(xw_r, degp_r, h0_r, y0_r, y1_r, y2_r):
    xw = xw_r[...]
    d = degp_r[...]
    h0_r[...] = xw[:, :D_OUT]
    ys = (y0_r, y1_r, y2_r)
    for j in range(N_HOP):
        dinv = lax.rsqrt(1.0 + d[j] + d[N_HOP + j])
        ys[j][...] = xw[:, (j + 1) * D_OUT:(j + 2) * D_OUT] * dinv[:, None]


def _mm2_call(xw, degp):
    oblk = pl.BlockSpec((RB, D_OUT), lambda i: (i, 0))
    y_sd = jax.ShapeDtypeStruct((N, WPAD), jnp.float32)
    return pl.pallas_call(
        _mm2_body,
        grid=(NRB,),
        in_specs=[pl.BlockSpec((RB, (N_HOP + 1) * D_OUT), lambda i: (i, 0)),
                  pl.BlockSpec((NC * N_HOP, RB), lambda i: (0, i))],
        out_specs=[oblk, oblk, oblk, oblk],
        out_shape=[jax.ShapeDtypeStruct((N, D_OUT), jnp.float32),
                   y_sd, y_sd, y_sd],
    )(xw, degp)


def _make_fin_body(j, last):
    def body(h_r, y_r, ap_r, degp_r, bias_r, out_r):
        d = degp_r[...]
        dinv = lax.rsqrt(1.0 + d[j] + d[N_HOP + j])
        agg = ap_r[0] + ap_r[1] + y_r[...]
        out = agg * dinv[:, None] + bias_r[...]
        h = h_r[...] + jnp.maximum(out, 0.0)
        if last:
            m = jnp.max(h, axis=1, keepdims=True)
            e = jnp.exp(h - m)
            h = h - m - jnp.log(jnp.sum(e, axis=1, keepdims=True))
        out_r[...] = h
    return body


def _fin_hop_call(h, y, aggp_j, degp, bias_j, j, last):
    oblk = pl.BlockSpec((RB, D_OUT), lambda i: (i, 0))
    return pl.pallas_call(
        _make_fin_body(j, last),
        grid=(NRB,),
        in_specs=[oblk, oblk,
                  pl.BlockSpec((NC, RB, WPAD), lambda i: (0, i, 0)),
                  pl.BlockSpec((NC * N_HOP, RB), lambda i: (0, i)),
                  pl.BlockSpec((1, D_OUT), lambda i: (0, 0))],
        out_specs=oblk,
        out_shape=jax.ShapeDtypeStruct((N, D_OUT), jnp.float32),
    )(h, y, aggp_j, degp, bias_j)


def kernel(bb0, bb1, bb2, bb3, edge0, edge1, edge2, comb_w, ego_W, ego_b,
           W0, b0, W1, b1, W2, b2):
    del bb0
    alphas = jax.nn.softmax(comb_w, axis=1)
    wall = jnp.stack([ego_W, W0, W1, W2], axis=0)
    t = alphas[:, :, None, None] * wall[:, None, :, :]
    wstack = t.transpose(1, 2, 0, 3).reshape(3, D_IN, (N_HOP + 1) * D_OUT)
    biascat = jnp.concatenate(
        [ego_b, jnp.zeros((N_HOP * D_OUT,), jnp.float32)]).reshape(1, -1)

    degp = _deg_kernel(edge0, edge1, edge2).reshape(NC * N_HOP, NPAD)
    xw = _mm1_call(bb1, bb2, bb3, wstack, biascat)
    h, y0, y1, y2 = _mm2_call(xw, degp)
    ys = (y0, y1, y2)
    edges = (edge0, edge1, edge2)
    biases = (b0.reshape(1, D_OUT), b1.reshape(1, D_OUT), b2.reshape(1, D_OUT))
    for j in range(N_HOP):
        aggp_j = _agg_one(ys[j], edges[j]).reshape(NC, NPAD, WPAD)
        h = _fin_hop_call(h, ys[j], aggp_j, degp, biases[j], j,
                          last=(j == N_HOP - 1))
    return h


# speedup vs baseline: 1.0943x; 1.0943x over previous
import functools

import jax
import jax.numpy as jnp
from jax import lax
from jax.experimental import pallas as pl
from jax.experimental.pallas import tpu as pltpu
from jax.experimental.pallas import tpu_sc as plsc

N = 10000
E = 320000
D_IN = 128
D_OUT = 64
N_HOP = 3

NC = 2
NS = 16
NW = NC * NS
EPW = E // NW
CHUNK = 128
NFULL = EPW // CHUNK
TAIL = EPW - NFULL * CHUNK
NPAD = 10240
STRIPE = NPAD // NS
WPAD = 64
RB = 512
NRB = NPAD // RB
NBUF = 6
LOOKA = 3

_mesh = plsc.VectorSubcoreMesh(core_axis_name="c", subcore_axis_name="s")


@functools.partial(
    pl.kernel,
    out_type=jax.ShapeDtypeStruct((NC * N_HOP * NPAD,), jnp.float32),
    mesh=_mesh,
    compiler_params=pltpu.CompilerParams(use_tc_tiling_on_sc=False),
    scratch_types=[
        pltpu.VMEM((EPW,), jnp.int32),
        pltpu.VMEM((CHUNK,), jnp.float32),
        pltpu.VMEM((STRIPE,), jnp.float32),
        pltpu.VMEM_SHARED((NPAD,), jnp.float32),
        pltpu.VMEM_SHARED((NPAD,), jnp.float32),
        pltpu.VMEM_SHARED((NPAD,), jnp.float32),
    ],
)
def _deg_kernel(edge0, edge1, edge2, degp_hbm, idx_v, ones_v, zeros_v,
                acc0, acc1, acc2):
    c = lax.axis_index("c")
    s = lax.axis_index("s")
    wid = s * NC + c
    edges = (edge0, edge1, edge2)
    accs = (acc0, acc1, acc2)
    for i in range(CHUNK // 16):
        ones_v[pl.ds(i * 16, 16)] = jnp.ones((16,), jnp.float32)

    def _zero_body(i, carry):
        zeros_v[pl.ds(i * 16, 16)] = jnp.zeros((16,), jnp.float32)
        return carry

    lax.fori_loop(0, STRIPE // 16, _zero_body, 0)
    for j in range(N_HOP):
        pltpu.sync_copy(zeros_v, accs[j].at[pl.ds(s * STRIPE, STRIPE)])
    plsc.subcore_barrier()

    for j in range(N_HOP):
        pltpu.sync_copy(edges[j].at[1, pl.ds(wid * EPW, EPW)], idx_v)

        def _body(ch, carry):
            pltpu.sync_copy(ones_v,
                            accs[j].at[idx_v.at[pl.ds(ch * CHUNK, CHUNK)]],
                            add=True)
            return carry

        lax.fori_loop(0, NFULL, _body, 0)
        pltpu.sync_copy(ones_v.at[pl.ds(0, TAIL)],
                        accs[j].at[idx_v.at[pl.ds(NFULL * CHUNK, TAIL)]],
                        add=True)
    plsc.subcore_barrier()
    for j in range(N_HOP):
        pltpu.sync_copy(
            accs[j].at[pl.ds(s * STRIPE, STRIPE)],
            degp_hbm.at[pl.ds((c * N_HOP + j) * NPAD + s * STRIPE, STRIPE)])


@functools.partial(
    pl.kernel,
    out_type=jax.ShapeDtypeStruct((NC * NPAD, WPAD), jnp.float32),
    mesh=_mesh,
    compiler_params=pltpu.CompilerParams(use_tc_tiling_on_sc=False),
    scratch_types=[
        pltpu.VMEM((EPW,), jnp.int32),
        pltpu.VMEM((EPW,), jnp.int32),
        [pltpu.VMEM((CHUNK, WPAD), jnp.float32) for _ in range(NBUF)],
        pltpu.VMEM((128, WPAD), jnp.float32),
        pltpu.VMEM_SHARED((NPAD, WPAD), jnp.float32),
        [pltpu.SemaphoreType.DMA for _ in range(NBUF)],
        [pltpu.SemaphoreType.DMA for _ in range(NBUF)],
    ],
)
def _agg_one(y_hbm, edge, aggp_hbm, sidx, didx, gbufs, zrow, acc,
             gsems, ssems):
    c = lax.axis_index("c")
    s = lax.axis_index("s")
    wid = s * NC + c

    def _zero_body(i, carry):
        for k in range(WPAD // 16):
            zrow[i, pl.ds(k * 16, 16)] = jnp.zeros((16,), jnp.float32)
        return carry

    lax.fori_loop(0, 128, _zero_body, 0)

    for t in range(STRIPE // 128):
        pltpu.sync_copy(zrow, acc.at[pl.ds(s * STRIPE + t * 128, 128)])
    pltpu.sync_copy(edge.at[0, pl.ds(wid * EPW, EPW)], sidx)
    pltpu.sync_copy(edge.at[1, pl.ds(wid * EPW, EPW)], didx)
    plsc.subcore_barrier()

    for b in range(LOOKA):
        pltpu.async_copy(y_hbm.at[sidx.at[pl.ds(b * CHUNK, CHUNK)]],
                         gbufs[b], gsems[b])

    def _group(g, carry):
        for b in range(NBUF):
            ch = g * NBUF + b
            ch_pf = ch + LOOKA
            slot_pf = (b + LOOKA) % NBUF

            @pl.when(jnp.logical_and(ch_pf >= NBUF, ch_pf < NFULL))
            def _drain():
                pltpu.make_async_copy(gbufs[slot_pf],
                                      acc.at[pl.ds(0, CHUNK)],
                                      ssems[slot_pf]).wait()

            @pl.when(ch_pf < NFULL)
            def _prefetch():
                pltpu.async_copy(
                    y_hbm.at[sidx.at[pl.ds(ch_pf * CHUNK, CHUNK)]],
                    gbufs[slot_pf], gsems[slot_pf])

            pltpu.make_async_copy(y_hbm.at[pl.ds(0, CHUNK)],
                                  gbufs[b], gsems[b]).wait()
            pltpu.async_copy(gbufs[b],
                             acc.at[didx.at[pl.ds(ch * CHUNK, CHUNK)]],
                             ssems[b], add=True)
        return carry

    lax.fori_loop(0, NFULL // NBUF, _group, 0)
    for b in range(NBUF):
        pltpu.make_async_copy(gbufs[b], acc.at[pl.ds(0, CHUNK)],
                              ssems[b]).wait()
    pltpu.async_copy(
        y_hbm.at[sidx.at[pl.ds(NFULL * CHUNK, TAIL)]],
        gbufs[0].at[pl.ds(0, TAIL)], gsems[0]).wait()
    pltpu.sync_copy(gbufs[0].at[pl.ds(0, TAIL)],
                    acc.at[didx.at[pl.ds(NFULL * CHUNK, TAIL)]],
                    add=True)
    plsc.subcore_barrier()
    pltpu.sync_copy(
        acc.at[pl.ds(s * STRIPE, STRIPE)],
        aggp_hbm.at[pl.ds(c * NPAD + s * STRIPE, STRIPE)])


def _mm1_body(bb1_r, bb2_r, bb3_r, wstack_r, biascat_r, xw_r):
    xw_r[...] = (
        jnp.dot(bb1_r[...], wstack_r[0], preferred_element_type=jnp.float32)
        + jnp.dot(bb2_r[...], wstack_r[1], preferred_element_type=jnp.float32)
        + jnp.dot(bb3_r[...], wstack_r[2], preferred_element_type=jnp.float32)
        + biascat_r[...])


def _mm1_call(bb1, bb2, bb3, wstack, biascat):
    blk = pl.BlockSpec((RB, D_IN), lambda i: (i, 0))
    return pl.pallas_call(
        _mm1_body,
        grid=(NRB,),
        in_specs=[blk, blk, blk,
                  pl.BlockSpec(wstack.shape, lambda i: (0, 0, 0)),
                  pl.BlockSpec(biascat.shape, lambda i: (0, 0))],
        out_specs=pl.BlockSpec((RB, (N_HOP + 1) * D_OUT), lambda i: (i, 0)),
        out_shape=jax.ShapeDtypeStruct((N, (N_HOP + 1) * D_OUT), jnp.float32),
    )(bb1, bb2, bb3, wstack, biascat)


def _mm2_body(xw_r, degp_r, h0_r, y0_r, y1_r, y2_r):
    xw = xw_r[...]
    d = degp_r[...]
    h0_r[...] = xw[:, :D_OUT]
    ys = (y0_r, y1_r, y2_r)
    for j in range(N_HOP):
        dinv = lax.rsqrt(1.0 + d[j] + d[N_HOP + j])
        ys[j][...] = xw[:, (j + 1) * D_OUT:(j + 2) * D_OUT] * dinv[:, None]


def _mm2_call(xw, degp):
    oblk = pl.BlockSpec((RB, D_OUT), lambda i: (i, 0))
    y_sd = jax.ShapeDtypeStruct((N, WPAD), jnp.float32)
    return pl.pallas_call(
        _mm2_body,
        grid=(NRB,),
        in_specs=[pl.BlockSpec((RB, (N_HOP + 1) * D_OUT), lambda i: (i, 0)),
                  pl.BlockSpec((NC * N_HOP, RB), lambda i: (0, i))],
        out_specs=[oblk, oblk, oblk, oblk],
        out_shape=[jax.ShapeDtypeStruct((N, D_OUT), jnp.float32),
                   y_sd, y_sd, y_sd],
    )(xw, degp)


def _fin_body(h_r, y0_r, y1_r, y2_r, a0_r, a1_r, a2_r, degp_r, bias_r, out_r):
    d = degp_r[...]
    h = h_r[...]
    ys = (y0_r, y1_r, y2_r)
    aps = (a0_r, a1_r, a2_r)
    for j in range(N_HOP):
        dinv = lax.rsqrt(1.0 + d[j] + d[N_HOP + j])
        agg = aps[j][0] + aps[j][1] + ys[j][...]
        out = agg * dinv[:, None] + bias_r[j][None, :]
        h = h + jnp.maximum(out, 0.0)
    m = jnp.max(h, axis=1, keepdims=True)
    e = jnp.exp(h - m)
    lse = jnp.log(jnp.sum(e, axis=1, keepdims=True))
    out_r[...] = h - m - lse


def _fin_call(h, y0, y1, y2, a0, a1, a2, degp, bias_h):
    oblk = pl.BlockSpec((RB, D_OUT), lambda i: (i, 0))
    ablk = pl.BlockSpec((NC, RB, WPAD), lambda i: (0, i, 0))
    return pl.pallas_call(
        _fin_body,
        grid=(NRB,),
        in_specs=[oblk, oblk, oblk, oblk, ablk, ablk, ablk,
                  pl.BlockSpec((NC * N_HOP, RB), lambda i: (0, i)),
                  pl.BlockSpec(bias_h.shape, lambda i: (0, 0))],
        out_specs=oblk,
        out_shape=jax.ShapeDtypeStruct((N, D_OUT), jnp.float32),
    )(h, y0, y1, y2, a0, a1, a2, degp, bias_h)


def kernel(bb0, bb1, bb2, bb3, edge0, edge1, edge2, comb_w, ego_W, ego_b,
           W0, b0, W1, b1, W2, b2):
    del bb0
    alphas = jax.nn.softmax(comb_w, axis=1)
    wall = jnp.stack([ego_W, W0, W1, W2], axis=0)
    t = alphas[:, :, None, None] * wall[:, None, :, :]
    wstack = t.transpose(1, 2, 0, 3).reshape(3, D_IN, (N_HOP + 1) * D_OUT)
    biascat = jnp.concatenate(
        [ego_b, jnp.zeros((N_HOP * D_OUT,), jnp.float32)]).reshape(1, -1)
    bias_h = jnp.stack([b0, b1, b2], axis=0)

    degp = _deg_kernel(edge0, edge1, edge2).reshape(NC * N_HOP, NPAD)
    xw = _mm1_call(bb1, bb2, bb3, wstack, biascat)
    h, y0, y1, y2 = _mm2_call(xw, degp)
    a0 = _agg_one(y0, edge0).reshape(NC, NPAD, WPAD)
    a1 = _agg_one(y1, edge1).reshape(NC, NPAD, WPAD)
    a2 = _agg_one(y2, edge2).reshape(NC, NPAD, WPAD)
    return _fin_call(h, y0, y1, y2, a0, a1, a2, degp, bias_h)
